# Initial kernel scaffold; baseline (speedup 1.0000x reference)
import jax
import jax.numpy as jnp
from jax.experimental import pallas as pl


def kernel(features, index_map, packpad_index, entity_type, keys_w, values_w):
    raise NotImplementedError("write your pallas kernel here")


# speedup vs baseline: 1.7391x; 1.7391x over previous
import functools

import jax
import jax.numpy as jnp
from jax import lax
from jax.experimental import pallas as pl
from jax.experimental.pallas import tpu as pltpu
from jax.experimental.pallas import tpu_sc as plsc

_B, _S, _N, _F = 16, 128, 2048, 8
_D = 64
_POSITIONS = 289
_EXTENT = 8.0
_STRIDE_Y = 17.0
_NW = 32
_PAIRS_PER_W = (_B * _S) // _NW
_L = 16


def _sc_body(feat_hbm, tok_hbm, et_hbm, keys_w, values_w,
             keys_out, vals_out,
             feat_v, et_v, pp_v, xrow, yrow, offrow, krow, vrow,
             kbuf, vbuf, ksem, vsem):
    wid = lax.axis_index("s") * 2 + lax.axis_index("c")
    b = wid // 2
    r0 = wid * _PAIRS_PER_W
    i_base = (wid % 2) * _PAIRS_PER_W

    pltpu.sync_copy(feat_hbm, feat_v)
    pltpu.sync_copy(et_hbm, et_v)
    pltpu.sync_copy(tok_hbm.at[b], pp_v)

    for c in range(_S // _L):
        tok = pp_v[pl.ds(c * _L, _L)]
        fbase = tok * _F
        xj = plsc.load_gather(feat_v, [fbase])
        yj = plsc.load_gather(feat_v, [fbase + 1])
        et = plsc.load_gather(et_v, [tok])
        xrow[pl.ds(c * _L, _L)] = xj
        yrow[pl.ds(c * _L, _L)] = yj
        offrow[pl.ds(c * _L, _L)] = et * _POSITIONS

    def body(i, carry):
        ib = jnp.full((_L,), i_base + i, jnp.int32)
        xi = plsc.load_gather(xrow, [ib])
        yi = plsc.load_gather(yrow, [ib])
        for c in range(_S // _L):
            sl = pl.ds(c * _L, _L)
            dx = jnp.minimum(jnp.maximum(xrow[sl] - xi, -_EXTENT), _EXTENT)
            dy = jnp.minimum(jnp.maximum(yrow[sl] - yi, -_EXTENT), _EXTENT)
            kf = (dx + _EXTENT) + _STRIDE_Y * (dy + _EXTENT)
            kt = kf.astype(jnp.int32)
            frac = kf - kt.astype(jnp.float32)
            bump = (frac > 0.5) | ((frac == 0.5) & ((kt & 1) == 1))
            ki = kt + bump.astype(jnp.int32)
            krow[sl] = ki
            vrow[sl] = ki + offrow[sl]
        ck = pltpu.async_copy(keys_w.at[krow], kbuf, ksem)
        cv = pltpu.async_copy(values_w.at[vrow], vbuf, vsem)
        ck.wait()
        cv.wait()
        r = r0 + i
        pltpu.sync_copy(kbuf, keys_out.at[pl.ds(r * _S, _S)])
        pltpu.sync_copy(vbuf, vals_out.at[pl.ds(r * _S, _S)])
        return carry

    lax.fori_loop(0, _PAIRS_PER_W, body, 0)


@jax.jit
def _sc_call(feat, tok, et, keys_w, values_w):
    mesh = plsc.VectorSubcoreMesh(core_axis_name="c", subcore_axis_name="s")
    f = pl.kernel(
        _sc_body,
        out_type=(
            jax.ShapeDtypeStruct((_B * _S * _S, _D), jnp.float32),
            jax.ShapeDtypeStruct((_B * _S * _S, _D), jnp.float32),
        ),
        mesh=mesh,
        compiler_params=pltpu.CompilerParams(
            needs_layout_passes=False, use_tc_tiling_on_sc=False),
        scratch_types=[
            pltpu.VMEM((_N * _F,), jnp.float32),
            pltpu.VMEM((_N,), jnp.int32),
            pltpu.VMEM((_S,), jnp.int32),
            pltpu.VMEM((_S,), jnp.float32),
            pltpu.VMEM((_S,), jnp.float32),
            pltpu.VMEM((_S,), jnp.int32),
            pltpu.VMEM((_S,), jnp.int32),
            pltpu.VMEM((_S,), jnp.int32),
            pltpu.VMEM((_S, _D), jnp.float32),
            pltpu.VMEM((_S, _D), jnp.float32),
            pltpu.SemaphoreType.DMA,
            pltpu.SemaphoreType.DMA,
        ],
    )
    return f(feat, tok, et, keys_w, values_w)


def kernel(features, index_map, packpad_index, entity_type, keys_w, values_w):
    tok = jnp.take(index_map, packpad_index, axis=0).astype(jnp.int32)
    keys_f, vals_f = _sc_call(features.reshape(_N * _F), tok,
                              entity_type.astype(jnp.int32).reshape(_N),
                              keys_w, values_w)
    return (keys_f.reshape(_B, _S, _S, _D), vals_f.reshape(_B, _S, _S, _D))
